# Initial kernel scaffold; baseline (speedup 1.0000x reference)
import jax
import jax.numpy as jnp
from jax.experimental import pallas as pl


def kernel(z_loga):
    raise NotImplementedError("write your pallas kernel here")


# speedup vs baseline: 8.8217x; 8.8217x over previous
import functools

import jax
import jax.numpy as jnp
from jax import lax
from jax.experimental import pallas as pl
from jax.experimental.pallas import tpu as pltpu
from jax.experimental.pallas import tpu_sc as plsc

_TEMPERATURE = 2.0 / 3.0
_MAGICAL_NUMBER = 0.8
_NUM_LAYERS = 32
_MASK_SIZE = 11008
_NUM_ZEROS = _MASK_SIZE - _MASK_SIZE // 2

_L = 16
_CHUNKS = _MASK_SIZE // _L

_mesh = plsc.VectorSubcoreMesh(core_axis_name="c", subcore_axis_name="s")


@functools.partial(
    pl.kernel,
    out_type=jax.ShapeDtypeStruct((_NUM_LAYERS, _MASK_SIZE), jnp.int32),
    mesh=_mesh,
    scratch_types=[pltpu.VMEM((_MASK_SIZE,), jnp.int32)],
    compiler_params=pltpu.CompilerParams(needs_layout_passes=False),
)
def _mask_rows(s_hbm, out_hbm, s_v):
    row = lax.axis_index("s") * 2 + lax.axis_index("c")
    pltpu.sync_copy(s_hbm.at[row], s_v)

    def _bits(i):
        return s_v[pl.ds(i * _L, _L)]

    lo0 = jnp.int32(0)
    hi0 = jnp.int32(0x3F800000)

    def count_le(t):
        def body(i, acc):
            return acc + jnp.where(_bits(i) <= t, 1, 0).astype(jnp.int32)
        acc = lax.fori_loop(0, _CHUNKS, body, jnp.zeros((_L,), jnp.int32))
        return jnp.sum(acc)

    def bs_cond(state):
        lo, hi = state
        return lo < hi

    def bs_body(state):
        lo, hi = state
        mid = lo + (hi - lo) // 2
        ge = count_le(mid) >= _NUM_ZEROS
        return jnp.where(ge, lo, mid + 1), jnp.where(ge, mid, hi)

    t, _ = lax.while_loop(bs_cond, bs_body, (lo0, hi0))

    def lt_body(i, acc):
        return acc + jnp.where(_bits(i) < t, 1, 0).astype(jnp.int32)

    c_lt = jnp.sum(lax.fori_loop(0, _CHUNKS, lt_body, jnp.zeros((_L,), jnp.int32)))
    needed = _NUM_ZEROS - c_lt

    def fin_body(i, cnt):
        v = s_v[pl.ds(i * _L, _L)]
        lt = v < t
        eq = v == t
        eqi = jnp.where(eq, 1, 0).astype(jnp.int32)
        tie_rank = cnt + jnp.cumsum(eqi)
        zero = lt | (eq & (tie_rank <= needed))
        s_v[pl.ds(i * _L, _L)] = jnp.where(zero, 0, v)
        return cnt + jnp.sum(eqi)

    lax.fori_loop(0, _CHUNKS, fin_body, jnp.int32(0))
    pltpu.sync_copy(s_v, out_hbm.at[row])


def kernel(z_loga):
    s = jax.nn.sigmoid(z_loga / _TEMPERATURE * _MAGICAL_NUMBER)
    sb = lax.bitcast_convert_type(s, jnp.int32)
    return lax.bitcast_convert_type(_mask_rows(sb), jnp.float32)


# speedup vs baseline: 29.2435x; 3.3149x over previous
import functools

import jax
import jax.numpy as jnp
from jax import lax
from jax.experimental import pallas as pl
from jax.experimental.pallas import tpu as pltpu
from jax.experimental.pallas import tpu_sc as plsc

_TEMPERATURE = 2.0 / 3.0
_MAGICAL_NUMBER = 0.8
_NUM_LAYERS = 32
_MASK_SIZE = 11008
_NUM_ZEROS = _MASK_SIZE - _MASK_SIZE // 2

_L = 16
_UNROLL = 8
_CHUNKS = _MASK_SIZE // _L
_OUTER = _CHUNKS // _UNROLL

_mesh = plsc.VectorSubcoreMesh(core_axis_name="c", subcore_axis_name="s")


@functools.partial(
    pl.kernel,
    out_type=jax.ShapeDtypeStruct((_NUM_LAYERS, _MASK_SIZE), jnp.int32),
    mesh=_mesh,
    scratch_types=[pltpu.VMEM((_MASK_SIZE,), jnp.int32)],
    compiler_params=pltpu.CompilerParams(needs_layout_passes=False),
)
def _mask_rows(s_hbm, out_hbm, s_v):
    row = lax.axis_index("s") * 2 + lax.axis_index("c")
    pltpu.sync_copy(s_hbm.at[row], s_v)

    def _bits(i, j):
        return s_v[pl.ds((i * _UNROLL + j) * _L, _L)]

    def mm_body(i, carry):
        mn, mx = carry
        for j in range(_UNROLL):
            v = _bits(i, j)
            mn = jnp.minimum(mn, v)
            mx = jnp.maximum(mx, v)
        return mn, mx

    mn0 = jnp.full((_L,), 0x3F800000, jnp.int32)
    mx0 = jnp.zeros((_L,), jnp.int32)
    mn, mx = lax.fori_loop(0, _OUTER, mm_body, (mn0, mx0))
    lo0 = jnp.min(mn)
    hi0 = jnp.max(mx)

    def count_le(t):
        def body(i, acc):
            for j in range(_UNROLL):
                acc = acc + jnp.where(_bits(i, j) <= t, 1, 0).astype(jnp.int32)
            return acc
        acc = lax.fori_loop(0, _OUTER, body, jnp.zeros((_L,), jnp.int32))
        return jnp.sum(acc)

    def bs_cond(state):
        lo, hi, _ = state
        return lo < hi

    def bs_body(state):
        lo, hi, c_lo = state
        mid = lo + (hi - lo) // 2
        c = count_le(mid)
        ge = c >= _NUM_ZEROS
        return (jnp.where(ge, lo, mid + 1),
                jnp.where(ge, mid, hi),
                jnp.where(ge, c_lo, c))

    t, _, c_lt = lax.while_loop(bs_cond, bs_body, (lo0, hi0, jnp.int32(0)))
    needed = _NUM_ZEROS - c_lt

    def fin_body(i, cnt):
        for j in range(_UNROLL):
            v = _bits(i, j)
            lt = v < t
            eq = v == t
            eqi = jnp.where(eq, 1, 0).astype(jnp.int32)
            tie_rank = cnt + jnp.cumsum(eqi)
            zero = lt | (eq & (tie_rank <= needed))
            s_v[pl.ds((i * _UNROLL + j) * _L, _L)] = jnp.where(zero, 0, v)
            cnt = cnt + jnp.sum(eqi)
        return cnt

    lax.fori_loop(0, _OUTER, fin_body, jnp.int32(0))
    pltpu.sync_copy(s_v, out_hbm.at[row])


def kernel(z_loga):
    s = jax.nn.sigmoid(z_loga / _TEMPERATURE * _MAGICAL_NUMBER)
    sb = lax.bitcast_convert_type(s, jnp.int32)
    return lax.bitcast_convert_type(_mask_rows(sb), jnp.float32)


# speedup vs baseline: 29.7955x; 1.0189x over previous
import functools

import jax
import jax.numpy as jnp
from jax import lax
from jax.experimental import pallas as pl
from jax.experimental.pallas import tpu as pltpu
from jax.experimental.pallas import tpu_sc as plsc

_TEMPERATURE = 2.0 / 3.0
_MAGICAL_NUMBER = 0.8
_NUM_LAYERS = 32
_MASK_SIZE = 11008
_NUM_ZEROS = _MASK_SIZE - _MASK_SIZE // 2

_L = 16
_UNROLL = 8
_CHUNKS = _MASK_SIZE // _L
_OUTER = _CHUNKS // _UNROLL

_mesh = plsc.VectorSubcoreMesh(core_axis_name="c", subcore_axis_name="s")


@functools.partial(
    pl.kernel,
    out_type=jax.ShapeDtypeStruct((_NUM_LAYERS, _MASK_SIZE), jnp.int32),
    mesh=_mesh,
    scratch_types=[pltpu.VMEM((_MASK_SIZE,), jnp.int32)],
    compiler_params=pltpu.CompilerParams(needs_layout_passes=False),
)
def _mask_rows(s_hbm, out_hbm, s_v):
    row = lax.axis_index("s") * 2 + lax.axis_index("c")
    pltpu.sync_copy(s_hbm.at[row], s_v)

    def _bits(i, j):
        return s_v[pl.ds((i * _UNROLL + j) * _L, _L)]

    _NACC = 4

    def mm_body(i, carry):
        mns, mxs = list(carry[0]), list(carry[1])
        for j in range(_UNROLL):
            v = _bits(i, j)
            a = j % _NACC
            mns[a] = jnp.minimum(mns[a], v)
            mxs[a] = jnp.maximum(mxs[a], v)
        return tuple(mns), tuple(mxs)

    mn0 = tuple(jnp.full((_L,), 0x3F800000, jnp.int32) for _ in range(_NACC))
    mx0 = tuple(jnp.zeros((_L,), jnp.int32) for _ in range(_NACC))
    mns, mxs = lax.fori_loop(0, _OUTER, mm_body, (mn0, mx0))
    lo0 = jnp.min(functools.reduce(jnp.minimum, mns))
    hi0 = jnp.max(functools.reduce(jnp.maximum, mxs))

    def count_le(t):
        def body(i, accs):
            accs = list(accs)
            for j in range(_UNROLL):
                a = j % _NACC
                accs[a] = accs[a] + jnp.where(_bits(i, j) <= t, 1, 0).astype(jnp.int32)
            return tuple(accs)
        acc0 = tuple(jnp.zeros((_L,), jnp.int32) for _ in range(_NACC))
        accs = lax.fori_loop(0, _OUTER, body, acc0)
        return jnp.sum(functools.reduce(jnp.add, accs))

    def bs_cond(state):
        lo, hi, _ = state
        return lo < hi

    def bs_body(state):
        lo, hi, c_lo = state
        mid = lo + (hi - lo) // 2
        c = count_le(mid)
        ge = c >= _NUM_ZEROS
        return (jnp.where(ge, lo, mid + 1),
                jnp.where(ge, mid, hi),
                jnp.where(ge, c_lo, c))

    t, _, c_lt = lax.while_loop(bs_cond, bs_body, (lo0, hi0, jnp.int32(0)))
    needed = _NUM_ZEROS - c_lt

    def fin_body(i, cnt):
        for j in range(_UNROLL):
            v = _bits(i, j)
            lt = v < t
            eq = v == t
            eqi = jnp.where(eq, 1, 0).astype(jnp.int32)
            tie_rank = cnt + jnp.cumsum(eqi)
            zero = lt | (eq & (tie_rank <= needed))
            s_v[pl.ds((i * _UNROLL + j) * _L, _L)] = jnp.where(zero, 0, v)
            cnt = cnt + plsc.all_reduce_population_count(eq)
        return cnt

    lax.fori_loop(0, _OUTER, fin_body, jnp.zeros((_L,), jnp.int32))
    pltpu.sync_copy(s_v, out_hbm.at[row])


def kernel(z_loga):
    s = jax.nn.sigmoid(z_loga / _TEMPERATURE * _MAGICAL_NUMBER)
    sb = lax.bitcast_convert_type(s, jnp.int32)
    return lax.bitcast_convert_type(_mask_rows(sb), jnp.float32)


# speedup vs baseline: 32.4721x; 1.0898x over previous
import functools

import jax
import jax.numpy as jnp
from jax import lax
from jax.experimental import pallas as pl
from jax.experimental.pallas import tpu as pltpu
from jax.experimental.pallas import tpu_sc as plsc

_TEMPERATURE = 2.0 / 3.0
_MAGICAL_NUMBER = 0.8
_NUM_LAYERS = 32
_MASK_SIZE = 11008
_NUM_ZEROS = _MASK_SIZE - _MASK_SIZE // 2

_L = 16
_UNROLL = 8
_CHUNKS = _MASK_SIZE // _L
_OUTER = _CHUNKS // _UNROLL

_mesh = plsc.VectorSubcoreMesh(core_axis_name="c", subcore_axis_name="s")


@functools.partial(
    pl.kernel,
    out_type=jax.ShapeDtypeStruct((_NUM_LAYERS, _MASK_SIZE), jnp.float32),
    mesh=_mesh,
    scratch_types=[pltpu.VMEM((_MASK_SIZE,), jnp.float32)],
    compiler_params=pltpu.CompilerParams(needs_layout_passes=False),
)
def _mask_rows(s_hbm, out_hbm, s_v):
    row = lax.axis_index("s") * 2 + lax.axis_index("c")
    pltpu.sync_copy(s_hbm.at[row], s_v)

    def _bits(i, j):
        return plsc.bitcast(s_v[pl.ds((i * _UNROLL + j) * _L, _L)], jnp.int32)

    _NACC = 4

    def mm_body(i, carry):
        mns, mxs = list(carry[0]), list(carry[1])
        for j in range(_UNROLL):
            v = _bits(i, j)
            a = j % _NACC
            mns[a] = jnp.minimum(mns[a], v)
            mxs[a] = jnp.maximum(mxs[a], v)
        return tuple(mns), tuple(mxs)

    mn0 = tuple(jnp.full((_L,), 0x3F800000, jnp.int32) for _ in range(_NACC))
    mx0 = tuple(jnp.zeros((_L,), jnp.int32) for _ in range(_NACC))
    mns, mxs = lax.fori_loop(0, _OUTER, mm_body, (mn0, mx0))
    lo0 = jnp.min(functools.reduce(jnp.minimum, mns))
    hi0 = jnp.max(functools.reduce(jnp.maximum, mxs))

    def count_le(t):
        def body(i, accs):
            accs = list(accs)
            for j in range(_UNROLL):
                a = j % _NACC
                accs[a] = accs[a] + jnp.where(_bits(i, j) <= t, 1, 0).astype(jnp.int32)
            return tuple(accs)
        acc0 = tuple(jnp.zeros((_L,), jnp.int32) for _ in range(_NACC))
        accs = lax.fori_loop(0, _OUTER, body, acc0)
        return jnp.sum(functools.reduce(jnp.add, accs))

    def bs_cond(state):
        lo, hi, _ = state
        return lo < hi

    def bs_body(state):
        lo, hi, c_lo = state
        mid = lo + (hi - lo) // 2
        c = count_le(mid)
        ge = c >= _NUM_ZEROS
        return (jnp.where(ge, lo, mid + 1),
                jnp.where(ge, mid, hi),
                jnp.where(ge, c_lo, c))

    t, _, c_lt = lax.while_loop(bs_cond, bs_body, (lo0, hi0, jnp.int32(0)))
    needed = _NUM_ZEROS - c_lt

    def fin_body(i, cnt):
        for j in range(_UNROLL):
            v = _bits(i, j)
            lt = v < t
            eq = v == t
            eqi = jnp.where(eq, 1, 0).astype(jnp.int32)
            tie_rank = cnt + jnp.cumsum(eqi)
            zero = lt | (eq & (tie_rank <= needed))
            s_v[pl.ds((i * _UNROLL + j) * _L, _L)] = plsc.bitcast(
                jnp.where(zero, 0, v), jnp.float32)
            cnt = cnt + plsc.all_reduce_population_count(eq)
        return cnt

    lax.fori_loop(0, _OUTER, fin_body, jnp.zeros((_L,), jnp.int32))
    pltpu.sync_copy(s_v, out_hbm.at[row])


def kernel(z_loga):
    s = jax.nn.sigmoid(z_loga / _TEMPERATURE * _MAGICAL_NUMBER)
    return _mask_rows(s)
